# Initial kernel scaffold; baseline (speedup 1.0000x reference)
import jax
import jax.numpy as jnp
from jax.experimental import pallas as pl


def kernel(x, edge_index, edge_weight, W1, W2):
    raise NotImplementedError("write your pallas kernel here")


# speedup vs baseline: 4.1627x; 4.1627x over previous
import functools

import jax
import jax.numpy as jnp
from jax import lax
from jax.experimental import pallas as pl
from jax.experimental.pallas import tpu as pltpu
from jax.experimental.pallas import tpu_sc as plsc

N = 10000
NP = 10240
E = 320000
D = 128

NC = 2
NS = 16
NW = NC * NS
EPW = E // NW
C = 80
NCHUNK = EPW // C
RPT = NP // NS
ZR = 128
BUF = 128

_mesh = plsc.VectorSubcoreMesh(core_axis_name="c", subcore_axis_name="s")


@functools.partial(
    pl.kernel,
    out_type=jax.ShapeDtypeStruct((NC, NP, D), jnp.float32),
    mesh=_mesh,
    scratch_types=[
        pltpu.VMEM((C,), jnp.int32),
        pltpu.VMEM((C,), jnp.int32),
        pltpu.VMEM((C,), jnp.float32),
        pltpu.VMEM((BUF, D), jnp.float32),
        pltpu.VMEM_SHARED((NP, D), jnp.float32),
        pltpu.SemaphoreType.DMA,
    ],
)
def _sc_aggregate(s_hbm, src_hbm, dst_hbm, w_hbm, out_hbm,
                  src_v, dst_v, w_v, rows_v, acc_sh, sem):
    cid = lax.axis_index("c")
    sid = lax.axis_index("s")
    wid = cid * NS + sid

    zeros16 = jnp.zeros((16,), jnp.float32)

    def zero_row(i, carry):
        for q in range(D // 16):
            rows_v[i, pl.ds(q * 16, 16)] = zeros16
        return carry

    lax.fori_loop(0, ZR, zero_row, 0)

    for k in range(RPT // ZR):
        pltpu.sync_copy(rows_v.at[pl.ds(0, ZR)],
                        acc_sh.at[pl.ds(sid * RPT + k * ZR, ZR)])
    plsc.subcore_barrier()

    def chunk_body(j, carry):
        base = pl.multiple_of(wid * EPW + j * C, C)
        pltpu.sync_copy(src_hbm.at[pl.ds(base, C)], src_v)
        pltpu.sync_copy(dst_hbm.at[pl.ds(base, C)], dst_v)
        pltpu.sync_copy(w_hbm.at[pl.ds(base, C)], w_v)
        pltpu.async_copy(s_hbm.at[src_v], rows_v.at[pl.ds(0, C)], sem).wait()

        def blk_body(b, c2):
            wv = w_v[pl.ds(b * 16, 16)]
            for e in range(16):
                w = wv[e]
                r = b * 16 + e
                for q in range(D // 16):
                    sl = pl.ds(q * 16, 16)
                    rows_v[r, sl] = rows_v[r, sl] * w
            return c2

        lax.fori_loop(0, C // 16, blk_body, 0)
        pltpu.sync_copy(rows_v.at[pl.ds(0, C)], acc_sh.at[dst_v], add=True)
        return carry

    lax.fori_loop(0, NCHUNK, chunk_body, 0)
    plsc.subcore_barrier()

    pltpu.sync_copy(acc_sh.at[pl.ds(sid * RPT, RPT)],
                    out_hbm.at[cid].at[pl.ds(sid * RPT, RPT)])


_RB = 1000
_RBP = 640


def _mm_body(x_ref, w_ref, o_ref):
    o_ref[...] = jnp.dot(x_ref[...], w_ref[...],
                         preferred_element_type=jnp.float32)


def _matmul(x, w):
    return pl.pallas_call(
        _mm_body,
        grid=(N // _RB,),
        in_specs=[
            pl.BlockSpec((_RB, D), lambda i: (i, 0)),
            pl.BlockSpec((D, D), lambda i: (0, 0)),
        ],
        out_specs=pl.BlockSpec((_RB, D), lambda i: (i, 0)),
        out_shape=jax.ShapeDtypeStruct((N, D), jnp.float32),
    )(x, w)


def _tanh_mm_body(p_ref, w_ref, o_ref):
    h = jnp.tanh(p_ref[0] + p_ref[1])
    o_ref[...] = jnp.dot(h, w_ref[...], preferred_element_type=jnp.float32)


def _tanh_matmul(p, w):
    return pl.pallas_call(
        _tanh_mm_body,
        grid=(NP // _RBP,),
        in_specs=[
            pl.BlockSpec((NC, _RBP, D), lambda i: (0, i, 0)),
            pl.BlockSpec((D, D), lambda i: (0, 0)),
        ],
        out_specs=pl.BlockSpec((_RBP, D), lambda i: (i, 0)),
        out_shape=jax.ShapeDtypeStruct((NP, D), jnp.float32),
    )(p, w)


def _tanh_sum_body(p_ref, o_ref):
    o_ref[...] = jnp.tanh(p_ref[0] + p_ref[1])


def _tanh_sum(p):
    return pl.pallas_call(
        _tanh_sum_body,
        grid=(NP // _RBP,),
        in_specs=[pl.BlockSpec((NC, _RBP, D), lambda i: (0, i, 0))],
        out_specs=pl.BlockSpec((_RBP, D), lambda i: (i, 0)),
        out_shape=jax.ShapeDtypeStruct((NP, D), jnp.float32),
    )(p)


@jax.jit
def kernel(x, edge_index, edge_weight, W1, W2):
    src = edge_index[0]
    dst = edge_index[1]
    s1 = _matmul(x, W1)
    p1 = _sc_aggregate(s1, src, dst, edge_weight)
    s2 = _tanh_matmul(p1, W2)
    p2 = _sc_aggregate(s2, src, dst, edge_weight)
    return _tanh_sum(p2)[:N]


# speedup vs baseline: 4.1657x; 1.0007x over previous
import functools

import jax
import jax.numpy as jnp
from jax import lax
from jax.experimental import pallas as pl
from jax.experimental.pallas import tpu as pltpu
from jax.experimental.pallas import tpu_sc as plsc

N = 10000
NP = 10240
E = 320000
D = 128

NC = 2
NS = 16
NW = NC * NS
EPW = E // NW
C = 80
NCHUNK = 128
EPAD = NCHUNK * C
NB = 8
RPT = NP // NS
ZR = 16

_mesh = plsc.VectorSubcoreMesh(core_axis_name="c", subcore_axis_name="s")


@functools.partial(
    pl.kernel,
    out_type=jax.ShapeDtypeStruct((NC, NP, D), jnp.float32),
    mesh=_mesh,
    scratch_types=[
        pltpu.VMEM((2, NB, C), jnp.int32),
        pltpu.VMEM((2, NB, C), jnp.int32),
        pltpu.VMEM((2, NB, C), jnp.float32),
        pltpu.VMEM((C, D), jnp.float32),
        pltpu.VMEM((C, D), jnp.float32),
        pltpu.VMEM((C, D), jnp.float32),
        pltpu.VMEM((C, D), jnp.float32),
        pltpu.VMEM((ZR, D), jnp.float32),
        pltpu.VMEM_SHARED((NP, D), jnp.float32),
        pltpu.SemaphoreType.DMA,
        pltpu.SemaphoreType.DMA,
        pltpu.SemaphoreType.DMA,
        pltpu.SemaphoreType.DMA,
        pltpu.SemaphoreType.DMA,
        pltpu.SemaphoreType.DMA,
        pltpu.SemaphoreType.DMA,
        pltpu.SemaphoreType.DMA,
        pltpu.SemaphoreType.DMA,
    ],
)
def _sc_aggregate(s_hbm, src_hbm, dst_hbm, w_hbm, out_hbm,
                  src_bb, dst_bb, w_bb, rows0_v, rows1_v, rows2_v, rows3_v,
                  zero_v, acc_sh,
                  gsem0, gsem1, gsem2, gsem3,
                  ssem0, ssem1, ssem2, ssem3, rsem):
    cid = lax.axis_index("c")
    sid = lax.axis_index("s")
    wid = cid * NS + sid

    rows = (rows0_v, rows1_v, rows2_v, rows3_v)
    gsems = (gsem0, gsem1, gsem2, gsem3)
    ssems = (ssem0, ssem1, ssem2, ssem3)

    def refill_copies(kb, half):
        base = kb * NB
        return (
            (src_hbm.at[wid, pl.ds(base, NB)], src_bb.at[half]),
            (dst_hbm.at[wid, pl.ds(base, NB)], dst_bb.at[half]),
            (w_hbm.at[wid, pl.ds(base, NB)], w_bb.at[half]),
        )

    def refill_start(kb, half):
        for s, d in refill_copies(kb, half):
            pltpu.async_copy(s, d, rsem)

    def refill_drain(kb, half):
        for s, d in refill_copies(kb, half):
            pltpu.make_async_copy(s, d, rsem).wait()

    refill_start(0, 0)
    refill_start(1, 1)

    zeros16 = jnp.zeros((16,), jnp.float32)

    def zero_row(i, carry):
        for q in range(D // 16):
            zero_v[i, pl.ds(q * 16, 16)] = zeros16
        return carry

    lax.fori_loop(0, ZR, zero_row, 0)
    for k in range(RPT // ZR):
        pltpu.sync_copy(zero_v, acc_sh.at[pl.ds(sid * RPT + k * ZR, ZR)])

    refill_drain(0, 0)
    refill_drain(1, 1)
    plsc.subcore_barrier()

    def idx_pos(j):
        k = j // NB
        return k % 2, j - k * NB

    def start_gather(j, b):
        h, jj = idx_pos(j)
        pltpu.async_copy(s_hbm.at[src_bb.at[h, jj]], rows[b], gsems[b])

    def consume(j, b):
        h, jj = idx_pos(j)
        pltpu.make_async_copy(s_hbm.at[src_bb.at[h, jj]], rows[b],
                              gsems[b]).wait()
        rb = rows[b]

        def blk_body(blk, c2):
            wv = w_bb[h, jj, pl.ds(blk * 16, 16)]
            for e in range(16):
                w = wv[e]
                r = blk * 16 + e
                for q in range(D // 16):
                    sl = pl.ds(q * 16, 16)
                    rb[r, sl] = rb[r, sl] * w
            return c2

        lax.fori_loop(0, C // 16, blk_body, 0)
        pltpu.async_copy(rb, acc_sh.at[dst_bb.at[h, jj]], ssems[b], add=True)

    def drain_scatter(j, b):
        h, jj = idx_pos(j)
        pltpu.make_async_copy(rows[b], acc_sh.at[dst_bb.at[h, jj]],
                              ssems[b]).wait()

    start_gather(0, 0)
    start_gather(1, 1)

    def quad_body(t, carry):
        for b in range(4):
            j = 4 * t + b
            k = j // NB
            jj = j - k * NB

            @pl.when(jnp.logical_and(jj == 2, j + NB - 2 < NCHUNK))
            def _():
                refill_start(k + 1, (k + 1) % 2)

            @pl.when(jnp.logical_and(jj == NB - 2, j + 2 < NCHUNK))
            def _():
                refill_drain(k + 1, (k + 1) % 2)

            b2 = (b + 2) % 4

            @pl.when(j + 2 < NCHUNK)
            def _():
                @pl.when(j >= 2)
                def _():
                    drain_scatter(j, b2)

                start_gather(j + 2, b2)

            consume(j, b)
        return carry

    lax.fori_loop(0, NCHUNK // 4, quad_body, 0)
    for b in range(4):
        drain_scatter(NCHUNK - 1, b)
    plsc.subcore_barrier()

    pltpu.sync_copy(acc_sh.at[pl.ds(sid * RPT, RPT)],
                    out_hbm.at[cid].at[pl.ds(sid * RPT, RPT)])


_RB = 1000
_RBP = 640


def _mm_body(x_ref, w_ref, o_ref):
    o_ref[...] = jnp.dot(x_ref[...], w_ref[...],
                         preferred_element_type=jnp.float32)


def _matmul(x, w):
    return pl.pallas_call(
        _mm_body,
        grid=(N // _RB,),
        in_specs=[
            pl.BlockSpec((_RB, D), lambda i: (i, 0)),
            pl.BlockSpec((D, D), lambda i: (0, 0)),
        ],
        out_specs=pl.BlockSpec((_RB, D), lambda i: (i, 0)),
        out_shape=jax.ShapeDtypeStruct((N, D), jnp.float32),
    )(x, w)


def _tanh_mm_body(p_ref, w_ref, o_ref):
    h = jnp.tanh(p_ref[0] + p_ref[1])
    o_ref[...] = jnp.dot(h, w_ref[...], preferred_element_type=jnp.float32)


def _tanh_matmul(p, w):
    return pl.pallas_call(
        _tanh_mm_body,
        grid=(NP // _RBP,),
        in_specs=[
            pl.BlockSpec((NC, _RBP, D), lambda i: (0, i, 0)),
            pl.BlockSpec((D, D), lambda i: (0, 0)),
        ],
        out_specs=pl.BlockSpec((_RBP, D), lambda i: (i, 0)),
        out_shape=jax.ShapeDtypeStruct((NP, D), jnp.float32),
    )(p, w)


def _tanh_sum_body(p_ref, o_ref):
    o_ref[...] = jnp.tanh(p_ref[0] + p_ref[1])


def _tanh_sum(p):
    return pl.pallas_call(
        _tanh_sum_body,
        grid=(NP // _RBP,),
        in_specs=[pl.BlockSpec((NC, _RBP, D), lambda i: (0, i, 0))],
        out_specs=pl.BlockSpec((_RBP, D), lambda i: (i, 0)),
        out_shape=jax.ShapeDtypeStruct((NP, D), jnp.float32),
    )(p)


@jax.jit
def kernel(x, edge_index, edge_weight, W1, W2):
    pad = ((0, 0), (0, EPAD - EPW))
    src3 = jnp.pad(edge_index[0].reshape(NW, EPW), pad).reshape(NW, NCHUNK, C)
    dst3 = jnp.pad(edge_index[1].reshape(NW, EPW), pad).reshape(NW, NCHUNK, C)
    w3 = jnp.pad(edge_weight.reshape(NW, EPW), pad).reshape(NW, NCHUNK, C)
    s1 = _matmul(x, W1)
    p1 = _sc_aggregate(s1, src3, dst3, w3)
    s2 = _tanh_matmul(p1, W2)
    p2 = _sc_aggregate(s2, src3, dst3, w3)
    return _tanh_sum(p2)[:N]


# speedup vs baseline: 4.1676x; 1.0005x over previous
import functools

import jax
import jax.numpy as jnp
from jax import lax
from jax.experimental import pallas as pl
from jax.experimental.pallas import tpu as pltpu
from jax.experimental.pallas import tpu_sc as plsc

N = 10000
NP = 10240
E = 320000
D = 128

NC = 2
NS = 16
NW = NC * NS
EPW = E // NW
C = 80
NCHUNK = 128
EPAD = NCHUNK * C
NB = 8
RPT = NP // NS
ZR = 16

_mesh = plsc.VectorSubcoreMesh(core_axis_name="c", subcore_axis_name="s")


@functools.partial(
    pl.kernel,
    out_type=jax.ShapeDtypeStruct((NC, NP, D), jnp.float32),
    mesh=_mesh,
    scratch_types=[
        pltpu.VMEM((2, NB, C), jnp.int32),
        pltpu.VMEM((2, NB, C), jnp.int32),
        pltpu.VMEM((2, NB, C), jnp.float32),
        pltpu.VMEM((C, D), jnp.float32),
        pltpu.VMEM((C, D), jnp.float32),
        pltpu.VMEM((C, D), jnp.float32),
        pltpu.VMEM((C, D), jnp.float32),
        pltpu.VMEM((ZR, D), jnp.float32),
        pltpu.VMEM_SHARED((NP, D), jnp.float32),
        pltpu.SemaphoreType.DMA,
        pltpu.SemaphoreType.DMA,
        pltpu.SemaphoreType.DMA,
        pltpu.SemaphoreType.DMA,
        pltpu.SemaphoreType.DMA,
        pltpu.SemaphoreType.DMA,
        pltpu.SemaphoreType.DMA,
        pltpu.SemaphoreType.DMA,
        pltpu.SemaphoreType.DMA,
    ],
)
def _sc_aggregate(s_hbm, src_hbm, dst_hbm, w_hbm, out_hbm,
                  src_bb, dst_bb, w_bb, rows0_v, rows1_v, rows2_v, rows3_v,
                  zero_v, acc_sh,
                  gsem0, gsem1, gsem2, gsem3,
                  ssem0, ssem1, ssem2, ssem3, rsem):
    cid = lax.axis_index("c")
    sid = lax.axis_index("s")
    wid = cid * NS + sid

    rows = (rows0_v, rows1_v, rows2_v, rows3_v)
    gsems = (gsem0, gsem1, gsem2, gsem3)
    ssems = (ssem0, ssem1, ssem2, ssem3)

    def refill_copies(kb, half):
        base = kb * NB
        return (
            (src_hbm.at[wid, pl.ds(base, NB)], src_bb.at[half]),
            (dst_hbm.at[wid, pl.ds(base, NB)], dst_bb.at[half]),
            (w_hbm.at[wid, pl.ds(base, NB)], w_bb.at[half]),
        )

    def refill_start(kb, half):
        for s, d in refill_copies(kb, half):
            pltpu.async_copy(s, d, rsem)

    def refill_drain(kb, half):
        for s, d in refill_copies(kb, half):
            pltpu.make_async_copy(s, d, rsem).wait()

    refill_start(0, 0)
    refill_start(1, 1)

    zeros16 = jnp.zeros((16,), jnp.float32)

    def zero_row(i, carry):
        for q in range(D // 16):
            zero_v[i, pl.ds(q * 16, 16)] = zeros16
        return carry

    lax.fori_loop(0, ZR, zero_row, 0)
    for k in range(RPT // ZR):
        pltpu.sync_copy(zero_v, acc_sh.at[pl.ds(sid * RPT + k * ZR, ZR)])

    refill_drain(0, 0)
    refill_drain(1, 1)
    plsc.subcore_barrier()

    def idx_pos(j):
        k = j // NB
        return k % 2, j - k * NB

    HC = C // 2

    def gather_copies(j, b):
        h, jj = idx_pos(j)
        return (
            (s_hbm.at[src_bb.at[h, jj, pl.ds(0, HC)]],
             rows[b].at[pl.ds(0, HC)]),
            (s_hbm.at[src_bb.at[h, jj, pl.ds(HC, HC)]],
             rows[b].at[pl.ds(HC, HC)]),
        )

    def start_gather(j, b):
        for s, d in gather_copies(j, b):
            pltpu.async_copy(s, d, gsems[b])

    def consume(j, b):
        h, jj = idx_pos(j)
        for s, d in gather_copies(j, b):
            pltpu.make_async_copy(s, d, gsems[b]).wait()
        rb = rows[b]

        def blk_body(blk, c2):
            wv = w_bb[h, jj, pl.ds(blk * 16, 16)]
            for e in range(16):
                w = wv[e]
                r = blk * 16 + e
                for q in range(D // 16):
                    sl = pl.ds(q * 16, 16)
                    rb[r, sl] = rb[r, sl] * w
            return c2

        lax.fori_loop(0, C // 16, blk_body, 0)
        pltpu.async_copy(rb, acc_sh.at[dst_bb.at[h, jj]], ssems[b], add=True)

    def drain_scatter(j, b):
        h, jj = idx_pos(j)
        pltpu.make_async_copy(rows[b], acc_sh.at[dst_bb.at[h, jj]],
                              ssems[b]).wait()

    start_gather(0, 0)
    start_gather(1, 1)

    def quad_body(t, carry):
        for b in range(4):
            j = 4 * t + b
            k = j // NB
            jj = j - k * NB

            @pl.when(jnp.logical_and(jj == 2, j + NB - 2 < NCHUNK))
            def _():
                refill_start(k + 1, (k + 1) % 2)

            @pl.when(jnp.logical_and(jj == NB - 2, j + 2 < NCHUNK))
            def _():
                refill_drain(k + 1, (k + 1) % 2)

            b2 = (b + 2) % 4

            @pl.when(j + 2 < NCHUNK)
            def _():
                @pl.when(j >= 2)
                def _():
                    drain_scatter(j, b2)

                start_gather(j + 2, b2)

            consume(j, b)
        return carry

    lax.fori_loop(0, NCHUNK // 4, quad_body, 0)
    for b in range(4):
        drain_scatter(NCHUNK - 1, b)
    plsc.subcore_barrier()

    pltpu.sync_copy(acc_sh.at[pl.ds(sid * RPT, RPT)],
                    out_hbm.at[cid].at[pl.ds(sid * RPT, RPT)])


_RB = 1000
_RBP = 640


def _mm_body(x_ref, w_ref, o_ref):
    o_ref[...] = jnp.dot(x_ref[...], w_ref[...],
                         preferred_element_type=jnp.float32)


def _matmul(x, w):
    return pl.pallas_call(
        _mm_body,
        grid=(N // _RB,),
        in_specs=[
            pl.BlockSpec((_RB, D), lambda i: (i, 0)),
            pl.BlockSpec((D, D), lambda i: (0, 0)),
        ],
        out_specs=pl.BlockSpec((_RB, D), lambda i: (i, 0)),
        out_shape=jax.ShapeDtypeStruct((N, D), jnp.float32),
    )(x, w)


def _tanh_mm_body(p_ref, w_ref, o_ref):
    h = jnp.tanh(p_ref[0] + p_ref[1])
    o_ref[...] = jnp.dot(h, w_ref[...], preferred_element_type=jnp.float32)


def _tanh_matmul(p, w):
    return pl.pallas_call(
        _tanh_mm_body,
        grid=(NP // _RBP,),
        in_specs=[
            pl.BlockSpec((NC, _RBP, D), lambda i: (0, i, 0)),
            pl.BlockSpec((D, D), lambda i: (0, 0)),
        ],
        out_specs=pl.BlockSpec((_RBP, D), lambda i: (i, 0)),
        out_shape=jax.ShapeDtypeStruct((NP, D), jnp.float32),
    )(p, w)


def _tanh_sum_body(p_ref, o_ref):
    o_ref[...] = jnp.tanh(p_ref[0] + p_ref[1])


def _tanh_sum(p):
    return pl.pallas_call(
        _tanh_sum_body,
        grid=(NP // _RBP,),
        in_specs=[pl.BlockSpec((NC, _RBP, D), lambda i: (0, i, 0))],
        out_specs=pl.BlockSpec((_RBP, D), lambda i: (i, 0)),
        out_shape=jax.ShapeDtypeStruct((NP, D), jnp.float32),
    )(p)


@jax.jit
def kernel(x, edge_index, edge_weight, W1, W2):
    pad = ((0, 0), (0, EPAD - EPW))
    src3 = jnp.pad(edge_index[0].reshape(NW, EPW), pad).reshape(NW, NCHUNK, C)
    dst3 = jnp.pad(edge_index[1].reshape(NW, EPW), pad).reshape(NW, NCHUNK, C)
    w3 = jnp.pad(edge_weight.reshape(NW, EPW), pad).reshape(NW, NCHUNK, C)
    s1 = _matmul(x, W1)
    p1 = _sc_aggregate(s1, src3, dst3, w3)
    s2 = _tanh_matmul(p1, W2)
    p2 = _sc_aggregate(s2, src3, dst3, w3)
    return _tanh_sum(p2)[:N]


# speedup vs baseline: 4.1815x; 1.0033x over previous
import functools

import jax
import jax.numpy as jnp
from jax import lax
from jax.experimental import pallas as pl
from jax.experimental.pallas import tpu as pltpu
from jax.experimental.pallas import tpu_sc as plsc

N = 10000
NP = 10240
E = 320000
D = 128

NC = 2
NS = 16
NW = NC * NS
EPW = E // NW
C = 80
NCHUNK = 128
EPAD = NCHUNK * C
NB = 8
RPT = NP // NS

_mesh = plsc.VectorSubcoreMesh(core_axis_name="c", subcore_axis_name="s")


@functools.partial(
    pl.kernel,
    out_type=jax.ShapeDtypeStruct((NC, NP, D), jnp.float32),
    mesh=_mesh,
    scratch_types=[
        pltpu.VMEM((2, NB, C), jnp.int32),
        pltpu.VMEM((2, NB, C), jnp.int32),
        pltpu.VMEM((2, NB, C), jnp.float32),
        pltpu.VMEM((C, D), jnp.float32),
        pltpu.VMEM((C, D), jnp.float32),
        pltpu.VMEM((C, D), jnp.float32),
        pltpu.VMEM((C, D), jnp.float32),
        pltpu.VMEM_SHARED((NP, D), jnp.float32),
        pltpu.SemaphoreType.DMA,
        pltpu.SemaphoreType.DMA,
        pltpu.SemaphoreType.DMA,
        pltpu.SemaphoreType.DMA,
        pltpu.SemaphoreType.DMA,
        pltpu.SemaphoreType.DMA,
        pltpu.SemaphoreType.DMA,
        pltpu.SemaphoreType.DMA,
        pltpu.SemaphoreType.DMA,
    ],
)
def _sc_aggregate(s_hbm, src_hbm, dst_hbm, w_hbm, out_hbm,
                  src_bb, dst_bb, w_bb, rows0_v, rows1_v, rows2_v, rows3_v,
                  acc_sh,
                  gsem0, gsem1, gsem2, gsem3,
                  ssem0, ssem1, ssem2, ssem3, rsem):
    cid = lax.axis_index("c")
    sid = lax.axis_index("s")
    wid = cid * NS + sid

    rows = (rows0_v, rows1_v, rows2_v, rows3_v)
    gsems = (gsem0, gsem1, gsem2, gsem3)
    ssems = (ssem0, ssem1, ssem2, ssem3)

    def refill_copies(kb, half):
        base = kb * NB
        return (
            (src_hbm.at[wid, pl.ds(base, NB)], src_bb.at[half]),
            (dst_hbm.at[wid, pl.ds(base, NB)], dst_bb.at[half]),
            (w_hbm.at[wid, pl.ds(base, NB)], w_bb.at[half]),
        )

    def refill_start(kb, half):
        for s, d in refill_copies(kb, half):
            pltpu.async_copy(s, d, rsem)

    def refill_drain(kb, half):
        for s, d in refill_copies(kb, half):
            pltpu.make_async_copy(s, d, rsem).wait()

    refill_start(0, 0)
    refill_start(1, 1)

    zeros16 = jnp.zeros((16,), jnp.float32)

    def zero_row(i, carry):
        for q in range(D // 16):
            rows0_v[i, pl.ds(q * 16, 16)] = zeros16
        return carry

    lax.fori_loop(0, C, zero_row, 0)
    for k in range(RPT // C):
        pltpu.sync_copy(rows0_v, acc_sh.at[pl.ds(sid * RPT + k * C, C)])

    refill_drain(0, 0)
    refill_drain(1, 1)
    plsc.subcore_barrier()

    def idx_pos(j):
        k = j // NB
        return k % 2, j - k * NB

    HC = C // 2

    def gather_copies(j, b):
        h, jj = idx_pos(j)
        return (
            (s_hbm.at[src_bb.at[h, jj, pl.ds(0, HC)]],
             rows[b].at[pl.ds(0, HC)]),
            (s_hbm.at[src_bb.at[h, jj, pl.ds(HC, HC)]],
             rows[b].at[pl.ds(HC, HC)]),
        )

    def start_gather(j, b):
        for s, d in gather_copies(j, b):
            pltpu.async_copy(s, d, gsems[b])

    def consume(j, b):
        h, jj = idx_pos(j)
        for s, d in gather_copies(j, b):
            pltpu.make_async_copy(s, d, gsems[b]).wait()
        rb = rows[b]

        def blk_body(blk, c2):
            wv = w_bb[h, jj, pl.ds(blk * 16, 16)]
            for e in range(16):
                w = wv[e]
                r = blk * 16 + e
                for q in range(D // 16):
                    sl = pl.ds(q * 16, 16)
                    rb[r, sl] = rb[r, sl] * w
            return c2

        lax.fori_loop(0, C // 16, blk_body, 0)
        pltpu.async_copy(rb, acc_sh.at[dst_bb.at[h, jj]], ssems[b], add=True)

    def drain_scatter(j, b):
        h, jj = idx_pos(j)
        pltpu.make_async_copy(rows[b], acc_sh.at[dst_bb.at[h, jj]],
                              ssems[b]).wait()

    start_gather(0, 0)
    start_gather(1, 1)

    def quad_body(t, carry):
        for b in range(4):
            j = 4 * t + b
            k = j // NB
            jj = j - k * NB

            @pl.when(jnp.logical_and(jj == 2, j + NB - 2 < NCHUNK))
            def _():
                refill_start(k + 1, (k + 1) % 2)

            @pl.when(jnp.logical_and(jj == NB - 2, j + 2 < NCHUNK))
            def _():
                refill_drain(k + 1, (k + 1) % 2)

            b2 = (b + 2) % 4

            @pl.when(j + 2 < NCHUNK)
            def _():
                @pl.when(j >= 2)
                def _():
                    drain_scatter(j, b2)

                start_gather(j + 2, b2)

            consume(j, b)
        return carry

    lax.fori_loop(0, NCHUNK // 4, quad_body, 0)
    for b in range(4):
        drain_scatter(NCHUNK - 1, b)
    plsc.subcore_barrier()

    pltpu.sync_copy(acc_sh.at[pl.ds(sid * RPT, RPT)],
                    out_hbm.at[cid].at[pl.ds(sid * RPT, RPT)])


_RB = 1000
_RBP = 640


def _mm_body(x_ref, w_ref, o_ref):
    o_ref[...] = jnp.dot(x_ref[...], w_ref[...],
                         preferred_element_type=jnp.float32)


def _matmul(x, w):
    return pl.pallas_call(
        _mm_body,
        grid=(N // _RB,),
        in_specs=[
            pl.BlockSpec((_RB, D), lambda i: (i, 0)),
            pl.BlockSpec((D, D), lambda i: (0, 0)),
        ],
        out_specs=pl.BlockSpec((_RB, D), lambda i: (i, 0)),
        out_shape=jax.ShapeDtypeStruct((N, D), jnp.float32),
    )(x, w)


def _tanh_mm_body(p_ref, w_ref, o_ref):
    h = jnp.tanh(p_ref[0] + p_ref[1])
    o_ref[...] = jnp.dot(h, w_ref[...], preferred_element_type=jnp.float32)


def _tanh_matmul(p, w):
    return pl.pallas_call(
        _tanh_mm_body,
        grid=(NP // _RBP,),
        in_specs=[
            pl.BlockSpec((NC, _RBP, D), lambda i: (0, i, 0)),
            pl.BlockSpec((D, D), lambda i: (0, 0)),
        ],
        out_specs=pl.BlockSpec((_RBP, D), lambda i: (i, 0)),
        out_shape=jax.ShapeDtypeStruct((NP, D), jnp.float32),
    )(p, w)


def _tanh_sum_body(p_ref, o_ref):
    o_ref[...] = jnp.tanh(p_ref[0] + p_ref[1])


def _tanh_sum(p):
    return pl.pallas_call(
        _tanh_sum_body,
        grid=(NP // _RBP,),
        in_specs=[pl.BlockSpec((NC, _RBP, D), lambda i: (0, i, 0))],
        out_specs=pl.BlockSpec((_RBP, D), lambda i: (i, 0)),
        out_shape=jax.ShapeDtypeStruct((NP, D), jnp.float32),
    )(p)


@jax.jit
def kernel(x, edge_index, edge_weight, W1, W2):
    pad = ((0, 0), (0, EPAD - EPW))
    src3 = jnp.pad(edge_index[0].reshape(NW, EPW), pad).reshape(NW, NCHUNK, C)
    dst3 = jnp.pad(edge_index[1].reshape(NW, EPW), pad).reshape(NW, NCHUNK, C)
    w3 = jnp.pad(edge_weight.reshape(NW, EPW), pad).reshape(NW, NCHUNK, C)
    s1 = _matmul(x, W1)
    p1 = _sc_aggregate(s1, src3, dst3, w3)
    s2 = _tanh_matmul(p1, W2)
    p2 = _sc_aggregate(s2, src3, dst3, w3)
    return _tanh_sum(p2)[:N]
